# baseline (speedup 1.0000x reference)
import jax
import jax.numpy as jnp
from jax import lax
from jax.experimental import pallas as pl
from jax.experimental.pallas import tpu as pltpu
from jax.experimental.pallas import tpu_sc as plsc

PAIRS = [[1,0],[0,5],[1,3],[0,2],[1,7],[0,11],[1,1],[0,0],[1,9],[0,7],[1,4],[0,9],[1,12],[0,3],[1,6],[0,14],[1,2],[0,1],[1,15],[0,13],[1,8],[0,6],[1,10],[0,4],[1,5],[0,8],[1,14],[0,10],[1,13],[0,12],[1,11],[0,15]]

_A = [o for l, o in PAIRS if l == 1]
_B = [o for l, o in PAIRS if l == 0]


def _pack4(vals):
    acc = 0
    for i, v in enumerate(vals):
        acc |= v << (4 * i)
    return jnp.int32(acc - (1 << 32) if acc >= (1 << 31) else acc)


_D = 256


def _unpack(lo, hi, k):
    word = jnp.where(k < 8, lo, hi)
    return (word >> (4 * (k & 7))) & 15


def _body(l1_hbm, l0_hbm, out_hbm, sem):
    k = lax.axis_index("s")
    a = _unpack(_pack4(_A[:8]), _pack4(_A[8:]), k)
    b = _unpack(_pack4(_B[:8]), _pack4(_B[8:]), k)

    cps = [
        pltpu.async_copy(
            l1_hbm.at[pl.ds(a, 1)], out_hbm.at[pl.ds(k * 16 + j, 1)], sem
        )
        for j in range(8)
    ]
    cps.append(
        pltpu.async_copy(
            l0_hbm.at[pl.ds(b * 8, 8)], out_hbm.at[pl.ds(k * 16 + 8, 8)], sem
        )
    )
    for cp in cps:
        cp.wait()


def _make_sc_gather():
    return pl.kernel(
        _body,
        out_type=jax.ShapeDtypeStruct((256, _D), jnp.float32),
        mesh=plsc.VectorSubcoreMesh(
            core_axis_name="c",
            subcore_axis_name="s",
            num_cores=1,
            num_subcores=16,
        ),
        scratch_types=[
            pltpu.SemaphoreType.DMA,
        ],
    )


@jax.jit
def kernel(layer1, layer0):
    l1f = layer1.reshape(layer1.shape[0], _D)
    l0f = layer0.reshape(layer0.shape[0] * 8, _D)
    out = _make_sc_gather()(l1f, l0f)
    return out.reshape(32, 8, _D)


# speedup vs baseline: 1.4031x; 1.4031x over previous
import jax
import jax.numpy as jnp
from jax import lax
from jax.experimental import pallas as pl
from jax.experimental.pallas import tpu as pltpu
from jax.experimental.pallas import tpu_sc as plsc

PAIRS = [[1,0],[0,5],[1,3],[0,2],[1,7],[0,11],[1,1],[0,0],[1,9],[0,7],[1,4],[0,9],[1,12],[0,3],[1,6],[0,14],[1,2],[0,1],[1,15],[0,13],[1,8],[0,6],[1,10],[0,4],[1,5],[0,8],[1,14],[0,10],[1,13],[0,12],[1,11],[0,15]]

_A = [o for l, o in PAIRS if l == 1]
_B = [o for l, o in PAIRS if l == 0]


def _pack4(vals):
    acc = 0
    for i, v in enumerate(vals):
        acc |= v << (4 * i)
    return jnp.int32(acc - (1 << 32) if acc >= (1 << 31) else acc)


_D = 256


def _unpack(lo, hi, k):
    word = jnp.where(k < 8, lo, hi)
    return (word >> (4 * (k & 7))) & 15


def _body(l1_hbm, l0_hbm, out_hbm, row_v, buf_v, sem, out_sem):
    k = lax.axis_index("s")
    a = _unpack(_pack4(_A[:8]), _pack4(_A[8:]), k)
    b = _unpack(_pack4(_B[:8]), _pack4(_B[8:]), k)

    cp1 = pltpu.async_copy(l1_hbm.at[pl.ds(a, 1)], row_v, sem)
    cp0 = pltpu.async_copy(l0_hbm.at[pl.ds(b * 8, 8)], buf_v, sem)
    cp1.wait()
    sts = [
        pltpu.async_copy(row_v, out_hbm.at[pl.ds(k * 16 + j, 1)], out_sem)
        for j in range(8)
    ]
    cp0.wait()
    sts.append(
        pltpu.async_copy(buf_v, out_hbm.at[pl.ds(k * 16 + 8, 8)], out_sem)
    )
    for st in sts:
        st.wait()


def _make_sc_gather():
    return pl.kernel(
        _body,
        out_type=jax.ShapeDtypeStruct((256, _D), jnp.float32),
        mesh=plsc.VectorSubcoreMesh(
            core_axis_name="c",
            subcore_axis_name="s",
            num_cores=1,
            num_subcores=16,
        ),
        scratch_types=[
            pltpu.VMEM((1, _D), jnp.float32),
            pltpu.VMEM((8, _D), jnp.float32),
            pltpu.SemaphoreType.DMA,
            pltpu.SemaphoreType.DMA,
        ],
    )


@jax.jit
def kernel(layer1, layer0):
    l1f = layer1.reshape(layer1.shape[0], _D)
    l0f = layer0.reshape(layer0.shape[0] * 8, _D)
    out = _make_sc_gather()(l1f, l0f)
    return out.reshape(32, 8, _D)
